# baseline (speedup 1.0000x reference)
import functools

import jax
import jax.numpy as jnp
import numpy as np
from jax import lax
from jax.experimental import pallas as pl
from jax.experimental.pallas import tpu as pltpu
from jax.experimental.pallas import tpu_sc as plsc

NC = 2
NS = 16
NW = NC * NS
SPLIT = 5


def _node_proj_body(emb_ref, pos_ref, wsrc_ref, bsrc_ref, wdst_ref, bdst_ref,
                    p_ref, q_ref):
    chan = emb_ref.shape[1]
    e = emb_ref[...]
    blk = e.shape[0]
    posb = pos_ref[...]
    pad = jnp.zeros((blk, chan - posb.shape[1]), jnp.float32)
    p_ref[...] = jnp.concatenate([
        jnp.dot(e, wsrc_ref[...], preferred_element_type=jnp.float32)
        + bsrc_ref[...], -0.1 * posb, pad], axis=1)
    q_ref[...] = jnp.concatenate([
        jnp.dot(e, wdst_ref[...], preferred_element_type=jnp.float32)
        + bdst_ref[...], 0.1 * posb, pad], axis=1)


def _node_proj(emb, pos, w_src, b_src, w_dst, b_dst, blk):
    n, chan = emb.shape
    pc = pos.shape[1]
    grid = n // blk
    full = lambda i: (0, 0)
    return pl.pallas_call(
        _node_proj_body,
        grid=(grid,),
        in_specs=[
            pl.BlockSpec((blk, chan), lambda i: (i, 0)),
            pl.BlockSpec((blk, pc), lambda i: (i, 0)),
            pl.BlockSpec((chan, chan), full),
            pl.BlockSpec((1, chan), full),
            pl.BlockSpec((chan, chan), full),
            pl.BlockSpec((1, chan), full),
        ],
        out_specs=[
            pl.BlockSpec((blk, 2 * chan), lambda i: (i, 0)),
            pl.BlockSpec((blk, 2 * chan), lambda i: (i, 0)),
        ],
        out_shape=[
            jax.ShapeDtypeStruct((n, 2 * chan), jnp.float32),
            jax.ShapeDtypeStruct((n, 2 * chan), jnp.float32),
        ],
    )(emb, pos, w_src, b_src.reshape(1, chan), w_dst, b_dst.reshape(1, chan))


def _make_gather_body(ck):
    def body(src_hbm, dst_hbm, p_hbm, q_hbm, z01_hbm,
             a_hbm, r_hbm, zg_hbm,
             sidx, didx, pg0, pg1, qg0, qg1, zg0, zg1,
             av0, av1, rv0, rv1,
             g0s, g1s, w0s, w1s):
        cid = lax.axis_index("c")
        sid = lax.axis_index("s")
        wid = cid * NS + sid
        epw = src_hbm.shape[0] // NW
        base = wid * epw
        nch = epw // ck
        nh = nch // 2
        nv = pg0.shape[1] // 16

        pltpu.sync_copy(src_hbm.at[pl.ds(base, epw)], sidx)
        pltpu.sync_copy(dst_hbm.at[pl.ds(base, epw)], didx)

        def fire_g(t, pgv, qgv, zgv, sem):
            sl = pl.ds(t * ck, ck)
            pltpu.async_copy(p_hbm.at[sidx.at[sl]], pgv, sem)
            pltpu.async_copy(q_hbm.at[didx.at[sl]], qgv, sem)
            pltpu.async_copy(z01_hbm.at[didx.at[sl]], zgv, sem)

        def wait_g(pgv, qgv, zgv, sem):
            sl = pl.ds(0, ck)
            pltpu.make_async_copy(p_hbm.at[sidx.at[sl]], pgv, sem).wait()
            pltpu.make_async_copy(q_hbm.at[didx.at[sl]], qgv, sem).wait()
            pltpu.make_async_copy(z01_hbm.at[didx.at[sl]], zgv, sem).wait()

        chan = a_hbm.shape[1]
        nva = chan // 16

        def add_pq(pgv, qgv, av, rv):
            def abody(e, c):
                for j in range(nva):
                    s = pl.ds(j * 16, 16)
                    av[e, s] = pgv[e, s] + qgv[e, s]
                s = pl.ds(chan, 16)
                rv[e, :] = pgv[e, s] + qgv[e, s]
                return c
            lax.fori_loop(0, ck, abody, 0)

        def fire_w(t, av, rv, zgv, sem):
            off = base + t * ck
            pltpu.async_copy(av, a_hbm.at[pl.ds(off, ck)], sem)
            pltpu.async_copy(rv, r_hbm.at[pl.ds(off, ck)], sem)
            pltpu.async_copy(zgv, zg_hbm.at[pl.ds(off, ck)], sem)

        def wait_w(av, rv, zgv, sem):
            pltpu.make_async_copy(av, a_hbm.at[pl.ds(0, ck)], sem).wait()
            pltpu.make_async_copy(rv, r_hbm.at[pl.ds(0, ck)], sem).wait()
            pltpu.make_async_copy(zgv, zg_hbm.at[pl.ds(0, ck)], sem).wait()

        fire_g(0, pg0, qg0, zg0, g0s)

        def pair(t2, carry):
            ta = 2 * t2
            tb = ta + 1

            @pl.when(t2 > 0)
            def _():
                wait_w(av1, rv1, zg1, w1s)

            fire_g(tb, pg1, qg1, zg1, g1s)
            wait_g(pg0, qg0, zg0, g0s)
            add_pq(pg0, qg0, av0, rv0)
            fire_w(ta, av0, rv0, zg0, w0s)
            wait_g(pg1, qg1, zg1, g1s)
            add_pq(pg1, qg1, av1, rv1)

            @pl.when(ta + 2 < nch)
            def _():
                wait_w(av0, rv0, zg0, w0s)
                fire_g(ta + 2, pg0, qg0, zg0, g0s)

            fire_w(tb, av1, rv1, zg1, w1s)
            return carry

        lax.fori_loop(0, nh, pair, 0)
        if nch % 2:
            wait_g(pg0, qg0, zg0, g0s)
            add_pq(pg0, qg0, av0, rv0)
            fire_w(nch - 1, av0, rv0, zg0, w0s)
        wait_w(av0, rv0, zg0, w0s)
        wait_w(av1, rv1, zg1, w1s)

    return body


def _sc_gather(src, dst, p, q, z01, ck):
    e = src.shape[0]
    epw = e // NW
    pw = p.shape[1]
    chan = pw // 2
    zc = z01.shape[1]
    mesh = plsc.VectorSubcoreMesh(core_axis_name="c", subcore_axis_name="s",
                                  num_cores=NC, num_subcores=NS)
    f = functools.partial(
        pl.kernel,
        out_type=[
            jax.ShapeDtypeStruct((e, chan), jnp.float32),
            jax.ShapeDtypeStruct((e, 16), jnp.float32),
            jax.ShapeDtypeStruct((e, zc), z01.dtype),
        ],
        mesh=mesh,
        scratch_types=[
            pltpu.VMEM((epw,), jnp.int32),
            pltpu.VMEM((epw,), jnp.int32),
            pltpu.VMEM((ck, pw), jnp.float32),
            pltpu.VMEM((ck, pw), jnp.float32),
            pltpu.VMEM((ck, pw), jnp.float32),
            pltpu.VMEM((ck, pw), jnp.float32),
            pltpu.VMEM((ck, zc), z01.dtype),
            pltpu.VMEM((ck, zc), z01.dtype),
            pltpu.VMEM((ck, chan), jnp.float32),
            pltpu.VMEM((ck, chan), jnp.float32),
            pltpu.VMEM((ck, 16), jnp.float32),
            pltpu.VMEM((ck, 16), jnp.float32),
            pltpu.SemaphoreType.DMA,
            pltpu.SemaphoreType.DMA,
            pltpu.SemaphoreType.DMA,
            pltpu.SemaphoreType.DMA,
        ],
    )(_make_gather_body(ck))
    return f(src, dst, p, q, z01)


def _tc_gate_body(a_ref, r_ref, lab_ref, zg_ref, wl_ref, bl_ref, wg_ref,
                  bg_ref, p0_ref, p1_ref, p2_ref, p3_ref):
    chan = wl_ref.shape[1]
    h = chan // 2
    a = (a_ref[...]
         + jnp.dot(lab_ref[...], wl_ref[...], preferred_element_type=jnp.float32)
         + bl_ref[...])
    r = r_ref[...][:, :3]
    s = a * jax.nn.sigmoid(a)
    g = jnp.dot(s, wg_ref[...], preferred_element_type=jnp.float32) + bg_ref[...]
    g0 = g[:, :chan]
    g1 = g[:, chan:2 * chan]
    g2 = g[:, 2 * chan:]
    zw = zg_ref[...]
    zlo = lax.bitcast_convert_type(zw << 16, jnp.float32)
    zhi = lax.bitcast_convert_type(zw & jnp.int32(-65536), jnp.float32)

    def zblock(gidx, half):
        src = zlo if half == 0 else zhi
        return src[:, gidx * h:(gidx + 1) * h]

    for out_ref, gidx in ((p0_ref, 0), (p1_ref, 1), (p2_ref, 2), (p3_ref, 3)):
        if gidx == 0:
            out_ref[:, :h] = g0[:, :h] * zblock(0, 0)
            out_ref[:, h:] = g0[:, h:] * zblock(0, 1)
        else:
            rk = r[:, gidx - 1:gidx]
            out_ref[:, :h] = g1[:, :h] * zblock(gidx, 0) + g2[:, :h] * rk
            out_ref[:, h:] = g1[:, h:] * zblock(gidx, 1) + g2[:, h:] * rk


def _tc_gates(a, r, lab, zg, w_label, b_label, w_gate, b_gate, blk):
    e = a.shape[0]
    rw = r.shape[1]
    ed, chan = w_label.shape
    zc = zg.shape[1]
    grid = e // blk
    full = lambda i: (0, 0)
    row = lambda i: (i, 0)
    outs = [jax.ShapeDtypeStruct((e, chan), jnp.float32) for _ in range(4)]
    return pl.pallas_call(
        _tc_gate_body,
        grid=(grid,),
        in_specs=[
            pl.BlockSpec((blk, chan), row),
            pl.BlockSpec((blk, rw), row),
            pl.BlockSpec((blk, ed), row),
            pl.BlockSpec((blk, zc), row),
            pl.BlockSpec((ed, chan), full),
            pl.BlockSpec((1, chan), full),
            pl.BlockSpec((chan, 3 * chan), full),
            pl.BlockSpec((1, 3 * chan), full),
        ],
        out_specs=[pl.BlockSpec((blk, chan), row) for _ in range(4)],
        out_shape=outs,
    )(a, r, lab, zg, w_label, b_label.reshape(1, chan), w_gate,
      b_gate.reshape(1, 3 * chan))


def _make_scatter_body(ck):
    def body(src_hbm, p0_hbm, p1_hbm, p2_hbm, p3_hbm,
             i0_hbm, i1_hbm, i2_hbm, i3_hbm,
             o0_hbm, o1_hbm, o2_hbm, o3_hbm,
             idx0, idx1, u0, u1, acc, l0s, l1s, s0s, s1s):
        cid = lax.axis_index("c")
        sid = lax.axis_index("s")
        e = src_hbm.shape[0]
        n = i0_hbm.shape[0]
        epc = e // NS
        nch = epc // ck
        nh = nch // 2
        rps = (n // NS) & ~7
        tail = n - NS * rps

        def do_group(psi_hbm, init_hbm, out_hbm):
            base = sid * epc

            def fire_lu(t, idxv, uv, sem):
                off = base + t * ck
                pltpu.async_copy(src_hbm.at[pl.ds(off, ck)], idxv, sem)
                pltpu.async_copy(psi_hbm.at[pl.ds(off, ck)], uv, sem)

            def wait_lu(idxv, uv, sem):
                pltpu.make_async_copy(src_hbm.at[pl.ds(0, ck)], idxv,
                                      sem).wait()
                pltpu.make_async_copy(psi_hbm.at[pl.ds(0, ck)], uv,
                                      sem).wait()

            def wait_s(idxv, uv, sem):
                pltpu.make_async_copy(uv, acc.at[idxv], sem).wait()

            rows = pl.ds(sid * rps, rps)
            trows = pl.ds(NS * rps, tail)
            pltpu.sync_copy(init_hbm.at[rows], acc.at[rows])

            @pl.when(sid == NS - 1)
            def _():
                pltpu.sync_copy(init_hbm.at[trows], acc.at[trows])

            plsc.subcore_barrier()
            fire_lu(0, idx0, u0, l0s)

            def pair(t2, carry):
                ta = 2 * t2

                @pl.when(t2 > 0)
                def _():
                    wait_s(idx1, u1, s1s)

                fire_lu(ta + 1, idx1, u1, l1s)
                wait_lu(idx0, u0, l0s)
                pltpu.async_copy(u0, acc.at[idx0], s0s, add=True)

                @pl.when(ta + 2 < nch)
                def _():
                    wait_s(idx0, u0, s0s)
                    fire_lu(ta + 2, idx0, u0, l0s)

                wait_lu(idx1, u1, l1s)
                pltpu.async_copy(u1, acc.at[idx1], s1s, add=True)
                return carry

            lax.fori_loop(0, nh, pair, 0)
            if nch % 2:
                wait_lu(idx0, u0, l0s)
                pltpu.async_copy(u0, acc.at[idx0], s0s, add=True)
            wait_s(idx0, u0, s0s)
            wait_s(idx1, u1, s1s)
            plsc.subcore_barrier()
            pltpu.sync_copy(acc.at[rows], out_hbm.at[rows])

            @pl.when(sid == NS - 1)
            def _():
                pltpu.sync_copy(acc.at[trows], out_hbm.at[trows])

            plsc.subcore_barrier()

        @pl.when(cid == 0)
        def _():
            do_group(p0_hbm, i0_hbm, o0_hbm)
            do_group(p1_hbm, i1_hbm, o1_hbm)

        @pl.when(cid == 1)
        def _():
            do_group(p2_hbm, i2_hbm, o2_hbm)
            do_group(p3_hbm, i3_hbm, o3_hbm)

    return body


def _sc_scatter(src, psis, inits, ck):
    n, chan = inits[0].shape
    mesh = plsc.VectorSubcoreMesh(core_axis_name="c", subcore_axis_name="s",
                                  num_cores=NC, num_subcores=NS)
    out = jax.ShapeDtypeStruct((n, chan), jnp.float32)
    f = functools.partial(
        pl.kernel,
        out_type=[out, out, out, out],
        mesh=mesh,
        scratch_types=[
            pltpu.VMEM((ck,), jnp.int32),
            pltpu.VMEM((ck,), jnp.int32),
            pltpu.VMEM((ck, chan), jnp.float32),
            pltpu.VMEM((ck, chan), jnp.float32),
            pltpu.VMEM_SHARED((n, chan), jnp.float32),
            pltpu.SemaphoreType.DMA,
            pltpu.SemaphoreType.DMA,
            pltpu.SemaphoreType.DMA,
            pltpu.SemaphoreType.DMA,
        ],
    )(_make_scatter_body(ck))
    return f(src, *psis, *inits)


def kernel(graph, pos, z_0, z_1, emb, edgelabels,
           W_label, b_label, W_src, b_src, W_dst, b_dst, W_gate, b_gate):
    n, chan = z_0.shape
    src = graph[0]
    dst = graph[1]
    e = src.shape[0]
    eh = e // SPLIT

    px, qx = _node_proj(emb, pos, W_src, b_src, W_dst, b_dst, blk=1000)
    z01 = jnp.concatenate([z_0, z_1.reshape(n, 3 * chan)],
                          axis=1).astype(jnp.bfloat16)
    zwords = lax.bitcast_convert_type(
        z01.reshape(n, 4, chan // 2, 2), jnp.int32).reshape(n, 2 * chan)
    perm = np.concatenate([np.arange(0, chan, 2), np.arange(1, chan, 2)])
    perm3 = np.concatenate([b * chan + perm for b in range(3)])
    inv = np.argsort(perm)
    wg_p = W_gate[:, perm3]
    bg_p = b_gate[perm3]

    zeros = jnp.zeros((n, chan), jnp.float32)
    parts = (zeros, zeros, zeros, zeros)
    ck1 = 40
    ck2 = 80
    for h in range(SPLIT):
        sl = slice(h * eh, (h + 1) * eh)
        a_e, r_e, zg = _sc_gather(src[sl], dst[sl], px, qx, zwords, ck=ck1)
        psis = _tc_gates(a_e, r_e, edgelabels[sl], zg, W_label, b_label,
                         wg_p, bg_p, blk=640)
        parts = _sc_scatter(src[sl], psis, parts, ck=ck2)

    o0, o1, o2, o3 = parts
    out0 = o0[:, inv]
    out1 = jnp.stack([o1[:, inv], o2[:, inv], o3[:, inv]], axis=1)
    return (out0, out1)


# speedup vs baseline: 1.0117x; 1.0117x over previous
import functools

import jax
import jax.numpy as jnp
import numpy as np
from jax import lax
from jax.experimental import pallas as pl
from jax.experimental.pallas import tpu as pltpu
from jax.experimental.pallas import tpu_sc as plsc

NC = 2
NS = 16
NW = NC * NS
SPLIT = 2


def _node_proj_body(emb_ref, pos_ref, wsrc_ref, bsrc_ref, wdst_ref, bdst_ref,
                    p_ref, q_ref):
    chan = emb_ref.shape[1]
    e = emb_ref[...]
    blk = e.shape[0]
    posb = pos_ref[...]
    pad = jnp.zeros((blk, chan - posb.shape[1]), jnp.float32)
    p_ref[...] = jnp.concatenate([
        jnp.dot(e, wsrc_ref[...], preferred_element_type=jnp.float32)
        + bsrc_ref[...], -0.1 * posb, pad], axis=1)
    q_ref[...] = jnp.concatenate([
        jnp.dot(e, wdst_ref[...], preferred_element_type=jnp.float32)
        + bdst_ref[...], 0.1 * posb, pad], axis=1)


def _node_proj(emb, pos, w_src, b_src, w_dst, b_dst, blk):
    n, chan = emb.shape
    pc = pos.shape[1]
    grid = n // blk
    full = lambda i: (0, 0)
    return pl.pallas_call(
        _node_proj_body,
        grid=(grid,),
        in_specs=[
            pl.BlockSpec((blk, chan), lambda i: (i, 0)),
            pl.BlockSpec((blk, pc), lambda i: (i, 0)),
            pl.BlockSpec((chan, chan), full),
            pl.BlockSpec((1, chan), full),
            pl.BlockSpec((chan, chan), full),
            pl.BlockSpec((1, chan), full),
        ],
        out_specs=[
            pl.BlockSpec((blk, 2 * chan), lambda i: (i, 0)),
            pl.BlockSpec((blk, 2 * chan), lambda i: (i, 0)),
        ],
        out_shape=[
            jax.ShapeDtypeStruct((n, 2 * chan), jnp.float32),
            jax.ShapeDtypeStruct((n, 2 * chan), jnp.float32),
        ],
    )(emb, pos, w_src, b_src.reshape(1, chan), w_dst, b_dst.reshape(1, chan))


def _make_gather_body(ck):
    def body(src_hbm, dst_hbm, p_hbm, q_hbm, z01_hbm,
             a_hbm, r_hbm, zg_hbm,
             sidx, didx, pg0, pg1, qg0, qg1, zg0, zg1,
             av0, av1, rv0, rv1,
             g0s, g1s, w0s, w1s):
        cid = lax.axis_index("c")
        sid = lax.axis_index("s")
        wid = cid * NS + sid
        epw = src_hbm.shape[0] // NW
        base = wid * epw
        nch = epw // ck
        nh = nch // 2
        nv = pg0.shape[1] // 16

        pltpu.sync_copy(src_hbm.at[pl.ds(base, epw)], sidx)
        pltpu.sync_copy(dst_hbm.at[pl.ds(base, epw)], didx)

        def fire_g(t, pgv, qgv, zgv, sem):
            sl = pl.ds(t * ck, ck)
            pltpu.async_copy(p_hbm.at[sidx.at[sl]], pgv, sem)
            pltpu.async_copy(q_hbm.at[didx.at[sl]], qgv, sem)
            pltpu.async_copy(z01_hbm.at[didx.at[sl]], zgv, sem)

        def wait_g(pgv, qgv, zgv, sem):
            sl = pl.ds(0, ck)
            pltpu.make_async_copy(p_hbm.at[sidx.at[sl]], pgv, sem).wait()
            pltpu.make_async_copy(q_hbm.at[didx.at[sl]], qgv, sem).wait()
            pltpu.make_async_copy(z01_hbm.at[didx.at[sl]], zgv, sem).wait()

        chan = a_hbm.shape[1]
        nva = chan // 16

        def add_pq(pgv, qgv, av, rv):
            def abody(e, c):
                for j in range(nva):
                    s = pl.ds(j * 16, 16)
                    av[e, s] = pgv[e, s] + qgv[e, s]
                s = pl.ds(chan, 16)
                rv[e, :] = pgv[e, s] + qgv[e, s]
                return c
            lax.fori_loop(0, ck, abody, 0)

        def fire_w(t, av, rv, zgv, sem):
            off = base + t * ck
            pltpu.async_copy(av, a_hbm.at[pl.ds(off, ck)], sem)
            pltpu.async_copy(rv, r_hbm.at[pl.ds(off, ck)], sem)
            pltpu.async_copy(zgv, zg_hbm.at[pl.ds(off, ck)], sem)

        def wait_w(av, rv, zgv, sem):
            pltpu.make_async_copy(av, a_hbm.at[pl.ds(0, ck)], sem).wait()
            pltpu.make_async_copy(rv, r_hbm.at[pl.ds(0, ck)], sem).wait()
            pltpu.make_async_copy(zgv, zg_hbm.at[pl.ds(0, ck)], sem).wait()

        fire_g(0, pg0, qg0, zg0, g0s)

        def pair(t2, carry):
            ta = 2 * t2
            tb = ta + 1

            @pl.when(t2 > 0)
            def _():
                wait_w(av1, rv1, zg1, w1s)

            fire_g(tb, pg1, qg1, zg1, g1s)
            wait_g(pg0, qg0, zg0, g0s)
            add_pq(pg0, qg0, av0, rv0)
            fire_w(ta, av0, rv0, zg0, w0s)
            wait_g(pg1, qg1, zg1, g1s)
            add_pq(pg1, qg1, av1, rv1)

            @pl.when(ta + 2 < nch)
            def _():
                wait_w(av0, rv0, zg0, w0s)
                fire_g(ta + 2, pg0, qg0, zg0, g0s)

            fire_w(tb, av1, rv1, zg1, w1s)
            return carry

        lax.fori_loop(0, nh, pair, 0)
        if nch % 2:
            wait_g(pg0, qg0, zg0, g0s)
            add_pq(pg0, qg0, av0, rv0)
            fire_w(nch - 1, av0, rv0, zg0, w0s)
        wait_w(av0, rv0, zg0, w0s)
        wait_w(av1, rv1, zg1, w1s)

    return body


def _sc_gather(src, dst, p, q, z01, ck):
    e = src.shape[0]
    epw = e // NW
    pw = p.shape[1]
    chan = pw // 2
    zc = z01.shape[1]
    mesh = plsc.VectorSubcoreMesh(core_axis_name="c", subcore_axis_name="s",
                                  num_cores=NC, num_subcores=NS)
    f = functools.partial(
        pl.kernel,
        out_type=[
            jax.ShapeDtypeStruct((e, chan), jnp.float32),
            jax.ShapeDtypeStruct((e, 16), jnp.float32),
            jax.ShapeDtypeStruct((e, zc), z01.dtype),
        ],
        mesh=mesh,
        scratch_types=[
            pltpu.VMEM((epw,), jnp.int32),
            pltpu.VMEM((epw,), jnp.int32),
            pltpu.VMEM((ck, pw), jnp.float32),
            pltpu.VMEM((ck, pw), jnp.float32),
            pltpu.VMEM((ck, pw), jnp.float32),
            pltpu.VMEM((ck, pw), jnp.float32),
            pltpu.VMEM((ck, zc), z01.dtype),
            pltpu.VMEM((ck, zc), z01.dtype),
            pltpu.VMEM((ck, chan), jnp.float32),
            pltpu.VMEM((ck, chan), jnp.float32),
            pltpu.VMEM((ck, 16), jnp.float32),
            pltpu.VMEM((ck, 16), jnp.float32),
            pltpu.SemaphoreType.DMA,
            pltpu.SemaphoreType.DMA,
            pltpu.SemaphoreType.DMA,
            pltpu.SemaphoreType.DMA,
        ],
    )(_make_gather_body(ck))
    return f(src, dst, p, q, z01)


def _tc_gate_body(a_ref, r_ref, lab_ref, zg_ref, wl_ref, bl_ref, wg_ref,
                  bg_ref, p0_ref, p1_ref, p2_ref, p3_ref):
    chan = wl_ref.shape[1]
    h = chan // 2
    a = (a_ref[...]
         + jnp.dot(lab_ref[...], wl_ref[...], preferred_element_type=jnp.float32)
         + bl_ref[...])
    r = r_ref[...][:, :3]
    s = a * jax.nn.sigmoid(a)
    g = jnp.dot(s, wg_ref[...], preferred_element_type=jnp.float32) + bg_ref[...]
    g0 = g[:, :chan]
    g1 = g[:, chan:2 * chan]
    g2 = g[:, 2 * chan:]
    zw = zg_ref[...]
    zlo = lax.bitcast_convert_type(zw << 16, jnp.float32)
    zhi = lax.bitcast_convert_type(zw & jnp.int32(-65536), jnp.float32)

    def zblock(gidx, half):
        src = zlo if half == 0 else zhi
        return src[:, gidx * h:(gidx + 1) * h]

    for out_ref, gidx in ((p0_ref, 0), (p1_ref, 1), (p2_ref, 2), (p3_ref, 3)):
        if gidx == 0:
            out_ref[:, :h] = g0[:, :h] * zblock(0, 0)
            out_ref[:, h:] = g0[:, h:] * zblock(0, 1)
        else:
            rk = r[:, gidx - 1:gidx]
            out_ref[:, :h] = g1[:, :h] * zblock(gidx, 0) + g2[:, :h] * rk
            out_ref[:, h:] = g1[:, h:] * zblock(gidx, 1) + g2[:, h:] * rk


def _tc_gates(a, r, lab, zg, w_label, b_label, w_gate, b_gate, blk):
    e = a.shape[0]
    rw = r.shape[1]
    ed, chan = w_label.shape
    zc = zg.shape[1]
    grid = e // blk
    full = lambda i: (0, 0)
    row = lambda i: (i, 0)
    outs = [jax.ShapeDtypeStruct((e, chan), jnp.float32) for _ in range(4)]
    return pl.pallas_call(
        _tc_gate_body,
        grid=(grid,),
        in_specs=[
            pl.BlockSpec((blk, chan), row),
            pl.BlockSpec((blk, rw), row),
            pl.BlockSpec((blk, ed), row),
            pl.BlockSpec((blk, zc), row),
            pl.BlockSpec((ed, chan), full),
            pl.BlockSpec((1, chan), full),
            pl.BlockSpec((chan, 3 * chan), full),
            pl.BlockSpec((1, 3 * chan), full),
        ],
        out_specs=[pl.BlockSpec((blk, chan), row) for _ in range(4)],
        out_shape=outs,
    )(a, r, lab, zg, w_label, b_label.reshape(1, chan), w_gate,
      b_gate.reshape(1, 3 * chan))


def _make_scatter_body(ck):
    def body(src_hbm, p0_hbm, p1_hbm, p2_hbm, p3_hbm,
             i0_hbm, i1_hbm, i2_hbm, i3_hbm,
             o0_hbm, o1_hbm, o2_hbm, o3_hbm,
             idx0, idx1, u0, u1, acc, l0s, l1s, s0s, s1s):
        cid = lax.axis_index("c")
        sid = lax.axis_index("s")
        e = src_hbm.shape[0]
        n = i0_hbm.shape[0]
        epc = e // NS
        nch = epc // ck
        nh = nch // 2
        rps = (n // NS) & ~7
        tail = n - NS * rps

        def do_group(psi_hbm, init_hbm, out_hbm):
            base = sid * epc

            def fire_lu(t, idxv, uv, sem):
                off = base + t * ck
                pltpu.async_copy(src_hbm.at[pl.ds(off, ck)], idxv, sem)
                pltpu.async_copy(psi_hbm.at[pl.ds(off, ck)], uv, sem)

            def wait_lu(idxv, uv, sem):
                pltpu.make_async_copy(src_hbm.at[pl.ds(0, ck)], idxv,
                                      sem).wait()
                pltpu.make_async_copy(psi_hbm.at[pl.ds(0, ck)], uv,
                                      sem).wait()

            def wait_s(idxv, uv, sem):
                pltpu.make_async_copy(uv, acc.at[idxv], sem).wait()

            rows = pl.ds(sid * rps, rps)
            trows = pl.ds(NS * rps, tail)
            pltpu.sync_copy(init_hbm.at[rows], acc.at[rows])

            @pl.when(sid == NS - 1)
            def _():
                pltpu.sync_copy(init_hbm.at[trows], acc.at[trows])

            plsc.subcore_barrier()
            fire_lu(0, idx0, u0, l0s)

            def pair(t2, carry):
                ta = 2 * t2

                @pl.when(t2 > 0)
                def _():
                    wait_s(idx1, u1, s1s)

                fire_lu(ta + 1, idx1, u1, l1s)
                wait_lu(idx0, u0, l0s)
                pltpu.async_copy(u0, acc.at[idx0], s0s, add=True)

                @pl.when(ta + 2 < nch)
                def _():
                    wait_s(idx0, u0, s0s)
                    fire_lu(ta + 2, idx0, u0, l0s)

                wait_lu(idx1, u1, l1s)
                pltpu.async_copy(u1, acc.at[idx1], s1s, add=True)
                return carry

            lax.fori_loop(0, nh, pair, 0)
            if nch % 2:
                wait_lu(idx0, u0, l0s)
                pltpu.async_copy(u0, acc.at[idx0], s0s, add=True)
            wait_s(idx0, u0, s0s)
            wait_s(idx1, u1, s1s)
            plsc.subcore_barrier()
            pltpu.sync_copy(acc.at[rows], out_hbm.at[rows])

            @pl.when(sid == NS - 1)
            def _():
                pltpu.sync_copy(acc.at[trows], out_hbm.at[trows])

            plsc.subcore_barrier()

        @pl.when(cid == 0)
        def _():
            do_group(p0_hbm, i0_hbm, o0_hbm)
            do_group(p1_hbm, i1_hbm, o1_hbm)

        @pl.when(cid == 1)
        def _():
            do_group(p2_hbm, i2_hbm, o2_hbm)
            do_group(p3_hbm, i3_hbm, o3_hbm)

    return body


def _sc_scatter(src, psis, inits, ck):
    n, chan = inits[0].shape
    mesh = plsc.VectorSubcoreMesh(core_axis_name="c", subcore_axis_name="s",
                                  num_cores=NC, num_subcores=NS)
    out = jax.ShapeDtypeStruct((n, chan), jnp.float32)
    f = functools.partial(
        pl.kernel,
        out_type=[out, out, out, out],
        mesh=mesh,
        scratch_types=[
            pltpu.VMEM((ck,), jnp.int32),
            pltpu.VMEM((ck,), jnp.int32),
            pltpu.VMEM((ck, chan), jnp.float32),
            pltpu.VMEM((ck, chan), jnp.float32),
            pltpu.VMEM_SHARED((n, chan), jnp.float32),
            pltpu.SemaphoreType.DMA,
            pltpu.SemaphoreType.DMA,
            pltpu.SemaphoreType.DMA,
            pltpu.SemaphoreType.DMA,
        ],
    )(_make_scatter_body(ck))
    return f(src, *psis, *inits)


def kernel(graph, pos, z_0, z_1, emb, edgelabels,
           W_label, b_label, W_src, b_src, W_dst, b_dst, W_gate, b_gate):
    n, chan = z_0.shape
    src = graph[0]
    dst = graph[1]
    e = src.shape[0]
    eh = e // SPLIT

    px, qx = _node_proj(emb, pos, W_src, b_src, W_dst, b_dst, blk=1000)
    z01 = jnp.concatenate([z_0, z_1.reshape(n, 3 * chan)],
                          axis=1).astype(jnp.bfloat16)
    zwords = lax.bitcast_convert_type(
        z01.reshape(n, 4, chan // 2, 2), jnp.int32).reshape(n, 2 * chan)
    perm = np.concatenate([np.arange(0, chan, 2), np.arange(1, chan, 2)])
    perm3 = np.concatenate([b * chan + perm for b in range(3)])
    inv = np.argsort(perm)
    wg_p = W_gate[:, perm3]
    bg_p = b_gate[perm3]

    zeros = jnp.zeros((n, chan), jnp.float32)
    parts = (zeros, zeros, zeros, zeros)
    ck1 = 40
    ck2 = 80
    for h in range(SPLIT):
        sl = slice(h * eh, (h + 1) * eh)
        a_e, r_e, zg = _sc_gather(src[sl], dst[sl], px, qx, zwords, ck=ck1)
        psis = _tc_gates(a_e, r_e, edgelabels[sl], zg, W_label, b_label,
                         wg_p, bg_p, blk=640)
        parts = _sc_scatter(src[sl], psis, parts, ck=ck2)

    o0, o1, o2, o3 = parts
    out0 = o0[:, inv]
    out1 = jnp.stack([o1[:, inv], o2[:, inv], o3[:, inv]], axis=1)
    return (out0, out1)


# speedup vs baseline: 1.0286x; 1.0167x over previous
import functools

import jax
import jax.numpy as jnp
from jax import lax
from jax.experimental import pallas as pl
from jax.experimental.pallas import tpu as pltpu
from jax.experimental.pallas import tpu_sc as plsc

NC = 2
NS = 16
NW = NC * NS
SPLIT = 5


def _node_proj_body(emb_ref, pos_ref, wsrc_ref, bsrc_ref, wdst_ref, bdst_ref,
                    p_ref, q_ref):
    chan = emb_ref.shape[1]
    e = emb_ref[...]
    blk = e.shape[0]
    posb = pos_ref[...]
    pad = jnp.zeros((blk, chan - posb.shape[1]), jnp.float32)
    p_ref[...] = jnp.concatenate([
        jnp.dot(e, wsrc_ref[...], preferred_element_type=jnp.float32)
        + bsrc_ref[...], -0.1 * posb, pad], axis=1)
    q_ref[...] = jnp.concatenate([
        jnp.dot(e, wdst_ref[...], preferred_element_type=jnp.float32)
        + bdst_ref[...], 0.1 * posb, pad], axis=1)


def _node_proj(emb, pos, w_src, b_src, w_dst, b_dst, blk):
    n, chan = emb.shape
    pc = pos.shape[1]
    grid = n // blk
    full = lambda i: (0, 0)
    return pl.pallas_call(
        _node_proj_body,
        grid=(grid,),
        in_specs=[
            pl.BlockSpec((blk, chan), lambda i: (i, 0)),
            pl.BlockSpec((blk, pc), lambda i: (i, 0)),
            pl.BlockSpec((chan, chan), full),
            pl.BlockSpec((1, chan), full),
            pl.BlockSpec((chan, chan), full),
            pl.BlockSpec((1, chan), full),
        ],
        out_specs=[
            pl.BlockSpec((blk, 2 * chan), lambda i: (i, 0)),
            pl.BlockSpec((blk, 2 * chan), lambda i: (i, 0)),
        ],
        out_shape=[
            jax.ShapeDtypeStruct((n, 2 * chan), jnp.float32),
            jax.ShapeDtypeStruct((n, 2 * chan), jnp.float32),
        ],
    )(emb, pos, w_src, b_src.reshape(1, chan), w_dst, b_dst.reshape(1, chan))


def _make_gather_body(ck):
    def body(src_hbm, dst_hbm, p_hbm, q_hbm, z01_hbm,
             s_hbm, zg_hbm,
             sidx, didx, pg0, pg1, qg0, qg1, zg0, zg1,
             g0s, g1s, w0s, w1s):
        cid = lax.axis_index("c")
        sid = lax.axis_index("s")
        wid = cid * NS + sid
        epw = src_hbm.shape[0] // NW
        base = wid * epw
        nch = epw // ck
        nh = nch // 2
        nv = pg0.shape[1] // 16

        pltpu.sync_copy(src_hbm.at[pl.ds(base, epw)], sidx)
        pltpu.sync_copy(dst_hbm.at[pl.ds(base, epw)], didx)

        def fire_g(t, pgv, qgv, zgv, sem):
            sl = pl.ds(t * ck, ck)
            pltpu.async_copy(p_hbm.at[sidx.at[sl]], pgv, sem)
            pltpu.async_copy(q_hbm.at[didx.at[sl]], qgv, sem)
            pltpu.async_copy(z01_hbm.at[didx.at[sl]], zgv, sem)

        def wait_g(pgv, qgv, zgv, sem):
            sl = pl.ds(0, ck)
            pltpu.make_async_copy(p_hbm.at[sidx.at[sl]], pgv, sem).wait()
            pltpu.make_async_copy(q_hbm.at[didx.at[sl]], qgv, sem).wait()
            pltpu.make_async_copy(z01_hbm.at[didx.at[sl]], zgv, sem).wait()

        def add_pq(pgv, qgv):
            def abody(e, c):
                for j in range(nv):
                    s = pl.ds(j * 16, 16)
                    pgv[e, s] = pgv[e, s] + qgv[e, s]
                return c
            lax.fori_loop(0, ck, abody, 0)

        def fire_w(t, pgv, zgv, sem):
            off = base + t * ck
            pltpu.async_copy(pgv, s_hbm.at[pl.ds(off, ck)], sem)
            pltpu.async_copy(zgv, zg_hbm.at[pl.ds(off, ck)], sem)

        def wait_w(pgv, zgv, sem):
            pltpu.make_async_copy(pgv, s_hbm.at[pl.ds(0, ck)], sem).wait()
            pltpu.make_async_copy(zgv, zg_hbm.at[pl.ds(0, ck)], sem).wait()

        fire_g(0, pg0, qg0, zg0, g0s)

        def pair(t2, carry):
            ta = 2 * t2
            tb = ta + 1

            @pl.when(t2 > 0)
            def _():
                wait_w(pg1, zg1, w1s)

            fire_g(tb, pg1, qg1, zg1, g1s)
            wait_g(pg0, qg0, zg0, g0s)
            add_pq(pg0, qg0)
            fire_w(ta, pg0, zg0, w0s)
            wait_g(pg1, qg1, zg1, g1s)
            add_pq(pg1, qg1)

            @pl.when(ta + 2 < nch)
            def _():
                wait_w(pg0, zg0, w0s)
                fire_g(ta + 2, pg0, qg0, zg0, g0s)

            fire_w(tb, pg1, zg1, w1s)
            return carry

        lax.fori_loop(0, nh, pair, 0)
        if nch % 2:
            wait_g(pg0, qg0, zg0, g0s)
            add_pq(pg0, qg0)
            fire_w(nch - 1, pg0, zg0, w0s)
        wait_w(pg0, zg0, w0s)
        wait_w(pg1, zg1, w1s)

    return body


def _sc_gather(src, dst, p, q, z01, ck):
    e = src.shape[0]
    epw = e // NW
    pw = p.shape[1]
    zc = z01.shape[1]
    mesh = plsc.VectorSubcoreMesh(core_axis_name="c", subcore_axis_name="s",
                                  num_cores=NC, num_subcores=NS)
    f = functools.partial(
        pl.kernel,
        out_type=[
            jax.ShapeDtypeStruct((e, pw), jnp.float32),
            jax.ShapeDtypeStruct((e, zc), jnp.float32),
        ],
        mesh=mesh,
        scratch_types=[
            pltpu.VMEM((epw,), jnp.int32),
            pltpu.VMEM((epw,), jnp.int32),
            pltpu.VMEM((ck, pw), jnp.float32),
            pltpu.VMEM((ck, pw), jnp.float32),
            pltpu.VMEM((ck, pw), jnp.float32),
            pltpu.VMEM((ck, pw), jnp.float32),
            pltpu.VMEM((ck, zc), jnp.float32),
            pltpu.VMEM((ck, zc), jnp.float32),
            pltpu.SemaphoreType.DMA,
            pltpu.SemaphoreType.DMA,
            pltpu.SemaphoreType.DMA,
            pltpu.SemaphoreType.DMA,
        ],
    )(_make_gather_body(ck))
    return f(src, dst, p, q, z01)


def _tc_gate_body(s_ref, lab_ref, zg_ref, wl_ref, bl_ref, wg_ref,
                  bg_ref, p0_ref, p1_ref, p2_ref, p3_ref):
    chan = wl_ref.shape[1]
    sv = s_ref[...]
    a = (sv[:, :chan]
         + jnp.dot(lab_ref[...], wl_ref[...], preferred_element_type=jnp.float32)
         + bl_ref[...])
    r = sv[:, chan:chan + 3]
    s = a * jax.nn.sigmoid(a)
    g = jnp.dot(s, wg_ref[...], preferred_element_type=jnp.float32) + bg_ref[...]
    g0 = g[:, :chan]
    g1 = g[:, chan:2 * chan]
    g2 = g[:, 2 * chan:]
    zg = zg_ref[...]
    p0_ref[...] = g0 * zg[:, :chan]
    p1_ref[...] = g1 * zg[:, chan:2 * chan] + g2 * r[:, 0:1]
    p2_ref[...] = g1 * zg[:, 2 * chan:3 * chan] + g2 * r[:, 1:2]
    p3_ref[...] = g1 * zg[:, 3 * chan:] + g2 * r[:, 2:3]


def _tc_gates(s, lab, zg, w_label, b_label, w_gate, b_gate, blk):
    e, pw = s.shape
    ed, chan = w_label.shape
    zc = zg.shape[1]
    grid = e // blk
    full = lambda i: (0, 0)
    row = lambda i: (i, 0)
    outs = [jax.ShapeDtypeStruct((e, chan), jnp.float32) for _ in range(4)]
    return pl.pallas_call(
        _tc_gate_body,
        grid=(grid,),
        in_specs=[
            pl.BlockSpec((blk, pw), row),
            pl.BlockSpec((blk, ed), row),
            pl.BlockSpec((blk, zc), row),
            pl.BlockSpec((ed, chan), full),
            pl.BlockSpec((1, chan), full),
            pl.BlockSpec((chan, 3 * chan), full),
            pl.BlockSpec((1, 3 * chan), full),
        ],
        out_specs=[pl.BlockSpec((blk, chan), row) for _ in range(4)],
        out_shape=outs,
    )(s, lab, zg, w_label, b_label.reshape(1, chan), w_gate,
      b_gate.reshape(1, 3 * chan))


def _make_scatter_body(ck):
    def body(src_hbm, p0_hbm, p1_hbm, p2_hbm, p3_hbm,
             i0_hbm, i1_hbm, i2_hbm, i3_hbm,
             o0_hbm, o1_hbm, o2_hbm, o3_hbm,
             idx0, idx1, u0, u1, acc, l0s, l1s, s0s, s1s):
        cid = lax.axis_index("c")
        sid = lax.axis_index("s")
        e = src_hbm.shape[0]
        n = i0_hbm.shape[0]
        epc = e // NS
        nch = epc // ck
        nh = nch // 2
        rps = (n // NS) & ~7
        tail = n - NS * rps

        def do_group(psi_hbm, init_hbm, out_hbm):
            base = sid * epc

            def fire_lu(t, idxv, uv, sem):
                off = base + t * ck
                pltpu.async_copy(src_hbm.at[pl.ds(off, ck)], idxv, sem)
                pltpu.async_copy(psi_hbm.at[pl.ds(off, ck)], uv, sem)

            def wait_lu(idxv, uv, sem):
                pltpu.make_async_copy(src_hbm.at[pl.ds(0, ck)], idxv,
                                      sem).wait()
                pltpu.make_async_copy(psi_hbm.at[pl.ds(0, ck)], uv,
                                      sem).wait()

            def wait_s(idxv, uv, sem):
                pltpu.make_async_copy(uv, acc.at[idxv], sem).wait()

            rows = pl.ds(sid * rps, rps)
            trows = pl.ds(NS * rps, tail)
            pltpu.sync_copy(init_hbm.at[rows], acc.at[rows])

            @pl.when(sid == NS - 1)
            def _():
                pltpu.sync_copy(init_hbm.at[trows], acc.at[trows])

            plsc.subcore_barrier()
            fire_lu(0, idx0, u0, l0s)

            def pair(t2, carry):
                ta = 2 * t2

                @pl.when(t2 > 0)
                def _():
                    wait_s(idx1, u1, s1s)

                fire_lu(ta + 1, idx1, u1, l1s)
                wait_lu(idx0, u0, l0s)
                pltpu.async_copy(u0, acc.at[idx0], s0s, add=True)

                @pl.when(ta + 2 < nch)
                def _():
                    wait_s(idx0, u0, s0s)
                    fire_lu(ta + 2, idx0, u0, l0s)

                wait_lu(idx1, u1, l1s)
                pltpu.async_copy(u1, acc.at[idx1], s1s, add=True)
                return carry

            lax.fori_loop(0, nh, pair, 0)
            if nch % 2:
                wait_lu(idx0, u0, l0s)
                pltpu.async_copy(u0, acc.at[idx0], s0s, add=True)
            wait_s(idx0, u0, s0s)
            wait_s(idx1, u1, s1s)
            plsc.subcore_barrier()
            pltpu.sync_copy(acc.at[rows], out_hbm.at[rows])

            @pl.when(sid == NS - 1)
            def _():
                pltpu.sync_copy(acc.at[trows], out_hbm.at[trows])

            plsc.subcore_barrier()

        @pl.when(cid == 0)
        def _():
            do_group(p0_hbm, i0_hbm, o0_hbm)
            do_group(p1_hbm, i1_hbm, o1_hbm)

        @pl.when(cid == 1)
        def _():
            do_group(p2_hbm, i2_hbm, o2_hbm)
            do_group(p3_hbm, i3_hbm, o3_hbm)

    return body


def _sc_scatter(src, psis, inits, ck):
    n, chan = inits[0].shape
    mesh = plsc.VectorSubcoreMesh(core_axis_name="c", subcore_axis_name="s",
                                  num_cores=NC, num_subcores=NS)
    out = jax.ShapeDtypeStruct((n, chan), jnp.float32)
    f = functools.partial(
        pl.kernel,
        out_type=[out, out, out, out],
        mesh=mesh,
        scratch_types=[
            pltpu.VMEM((ck,), jnp.int32),
            pltpu.VMEM((ck,), jnp.int32),
            pltpu.VMEM((ck, chan), jnp.float32),
            pltpu.VMEM((ck, chan), jnp.float32),
            pltpu.VMEM_SHARED((n, chan), jnp.float32),
            pltpu.SemaphoreType.DMA,
            pltpu.SemaphoreType.DMA,
            pltpu.SemaphoreType.DMA,
            pltpu.SemaphoreType.DMA,
        ],
    )(_make_scatter_body(ck))
    return f(src, *psis, *inits)


def kernel(graph, pos, z_0, z_1, emb, edgelabels,
           W_label, b_label, W_src, b_src, W_dst, b_dst, W_gate, b_gate):
    n, chan = z_0.shape
    src = graph[0]
    dst = graph[1]
    e = src.shape[0]
    eh = e // SPLIT

    px, qx = _node_proj(emb, pos, W_src, b_src, W_dst, b_dst, blk=1000)
    z01 = jnp.concatenate([z_0, z_1.reshape(n, 3 * chan)], axis=1)

    zeros = jnp.zeros((n, chan), jnp.float32)
    parts = (zeros, zeros, zeros, zeros)
    ck1 = 40
    ck2 = 80
    for h in range(SPLIT):
        sl = slice(h * eh, (h + 1) * eh)
        s, zg = _sc_gather(src[sl], dst[sl], px, qx, z01, ck=ck1)
        psis = _tc_gates(s, edgelabels[sl], zg, W_label, b_label,
                         W_gate, b_gate, blk=640)
        parts = _sc_scatter(src[sl], psis, parts, ck=ck2)

    o0, o1, o2, o3 = parts
    out0 = o0
    out1 = jnp.stack([o1, o2, o3], axis=1)
    return (out0, out1)
